# baseline (speedup 1.0000x reference)
import functools

import jax
import jax.numpy as jnp
from jax import lax
from jax.experimental import pallas as pl
from jax.experimental.pallas import tpu as pltpu
from jax.experimental.pallas import tpu_sc as plsc

N_VOCAB = 1000000
EMB_DIM = 64
BATCH = 4096
SEQ_LEN = 200

B = BATCH * SEQ_LEN
NC = 2
NS = 16
NW = NC * NS
B_PER_W = B // NW
CHUNK = 800
N_CHUNKS = B_PER_W // CHUNK


def _make_gather():
    mesh = plsc.VectorSubcoreMesh(core_axis_name="c", subcore_axis_name="s")

    @functools.partial(
        pl.kernel,
        mesh=mesh,
        compiler_params=pltpu.CompilerParams(use_tc_tiling_on_sc=False),
        out_type=jax.ShapeDtypeStruct((B, EMB_DIM), jnp.float32),
        scratch_types=[
            pltpu.VMEM((CHUNK,), jnp.int32),
            pltpu.VMEM((CHUNK, EMB_DIM), jnp.float32),
            pltpu.SemaphoreType.DMA,
        ],
    )
    def gather_kernel(idx_hbm, table_hbm, out_hbm, idx_v, rows_v, sem):
        wid = lax.axis_index("s") * NC + lax.axis_index("c")
        base = wid * B_PER_W

        def body(c, carry):
            row0 = base + c * CHUNK
            pltpu.sync_copy(idx_hbm.at[pl.ds(row0, CHUNK)], idx_v)
            pltpu.async_copy(table_hbm.at[idx_v], rows_v, sem).wait()
            pltpu.sync_copy(rows_v, out_hbm.at[pl.ds(row0, CHUNK)])
            return carry

        lax.fori_loop(0, N_CHUNKS, body, 0)

    return gather_kernel


_gather = _make_gather()


def kernel(word_seq, emb_weight):
    flat_idx = word_seq.reshape(B)
    out = _gather(flat_idx, emb_weight)
    return out.reshape(BATCH, SEQ_LEN, EMB_DIM)


# speedup vs baseline: 1.0155x; 1.0155x over previous
import functools

import jax
import jax.numpy as jnp
from jax import lax
from jax.experimental import pallas as pl
from jax.experimental.pallas import tpu as pltpu
from jax.experimental.pallas import tpu_sc as plsc

N_VOCAB = 1000000
EMB_DIM = 64
BATCH = 4096
SEQ_LEN = 200

NC = 2
NS = 16
NW = NC * NS
ROWS_PER_W = BATCH // NW
R = 4
STEPS = ROWS_PER_W // R


def _make_gather():
    mesh = plsc.VectorSubcoreMesh(core_axis_name="c", subcore_axis_name="s")

    @functools.partial(
        pl.kernel,
        mesh=mesh,
        compiler_params=pltpu.CompilerParams(use_tc_tiling_on_sc=False),
        out_type=jax.ShapeDtypeStruct((BATCH, SEQ_LEN, EMB_DIM), jnp.float32),
        scratch_types=[
            pltpu.VMEM((2, R, SEQ_LEN), jnp.int32),
            pltpu.VMEM((2, R, SEQ_LEN, EMB_DIM), jnp.float32),
            pltpu.SemaphoreType.DMA,
            pltpu.SemaphoreType.DMA,
            pltpu.SemaphoreType.DMA,
            pltpu.SemaphoreType.DMA,
        ],
    )
    def gather_kernel(idx_hbm, table_hbm, out_hbm, idx_v, rows_v,
                      gsem0, gsem1, wsem0, wsem1):
        wid = lax.axis_index("s") * NC + lax.axis_index("c")
        base = wid * ROWS_PER_W
        gsems = (gsem0, gsem1)
        wsems = (wsem0, wsem1)

        def load_and_fire(step, buf):
            b0 = base + step * R
            pltpu.sync_copy(idx_hbm.at[pl.ds(b0, R)], idx_v.at[buf])
            for r in range(R):
                pltpu.async_copy(
                    table_hbm.at[idx_v.at[buf, r]], rows_v.at[buf, r],
                    gsems[buf])

        def wait_gathers(buf):
            for r in range(R):
                pltpu.make_async_copy(
                    table_hbm.at[idx_v.at[buf, r]], rows_v.at[buf, r],
                    gsems[buf]).wait()

        def fire_write(step, buf):
            b0 = base + step * R
            pltpu.async_copy(rows_v.at[buf], out_hbm.at[pl.ds(b0, R)],
                             wsems[buf])

        def wait_write(step, buf):
            b0 = base + step * R
            pltpu.make_async_copy(rows_v.at[buf], out_hbm.at[pl.ds(b0, R)],
                                  wsems[buf]).wait()

        load_and_fire(0, 0)

        def pair_body(i, carry):
            for k in (0, 1):
                c = 2 * i + k
                buf = k
                nbuf = 1 - k

                @pl.when(c + 1 < STEPS)
                def _():
                    @pl.when(c >= 1)
                    def _():
                        wait_write(c - 1, nbuf)
                    load_and_fire(c + 1, nbuf)

                wait_gathers(buf)
                fire_write(c, buf)
            return carry

        lax.fori_loop(0, STEPS // 2, pair_body, 0)
        wait_write(STEPS - 2, 0)
        wait_write(STEPS - 1, 1)

    return gather_kernel


_gather = _make_gather()


def kernel(word_seq, emb_weight):
    return _gather(word_seq, emb_weight)
